# baseline (speedup 1.0000x reference)
import functools

import jax
import jax.numpy as jnp
from jax import lax
from jax.experimental import pallas as pl
from jax.experimental.pallas import tpu as pltpu
from jax.experimental.pallas import tpu_sc as plsc

B, T = 4, 8192
VOCAB, CHAR_DIM, N_FREQ = 1000, 32, 4
OUT_DIM = CHAR_DIM * N_FREQ
NTOK = B * T
NUM_CORES, NUM_SUBCORES, LANES = 2, 16, 16
NW = NUM_CORES * NUM_SUBCORES
TPW = NTOK // NW
CHUNK = 128
TPOS = T // NW
NTC = TPOS // CHUNK
NU = NTC * B
ROWS_PER_SUB = 64

_mesh = plsc.VectorSubcoreMesh(core_axis_name="c", subcore_axis_name="s")


@functools.partial(
    pl.kernel,
    out_type=jax.ShapeDtypeStruct((NTOK, OUT_DIM), jnp.float32),
    mesh=_mesh,
    scratch_types=[
        [pltpu.VMEM((CHUNK,), jnp.int32) for _ in range(NU)],
        [pltpu.VMEM((CHUNK, OUT_DIM), jnp.float32) for _ in range(2)],
        [pltpu.VMEM((CHUNK, OUT_DIM), jnp.float32) for _ in range(2)],
        [pltpu.VMEM((ROWS_PER_SUB * CHAR_DIM,), jnp.float32)
         for _ in range(N_FREQ)],
        pltpu.VMEM((ROWS_PER_SUB, OUT_DIM), jnp.float32),
        pltpu.VMEM_SHARED((VOCAB, OUT_DIM), jnp.float32),
        [pltpu.SemaphoreType.DMA for _ in range(NU)],
        [pltpu.SemaphoreType.DMA for _ in range(2)],
        [pltpu.SemaphoreType.DMA for _ in range(2)],
        [pltpu.SemaphoreType.DMA for _ in range(2)],
    ],
)
def _encode(idx_hbm, emb0, emb1, emb2, emb3, pos_hbm, out_hbm,
            idx_v, pos_v, rows_v, tbl_v, cat_v, cat_sh, si, sp, sg, so):
    sid = lax.axis_index("s")
    w = sid * NUM_CORES + lax.axis_index("c")
    t_base = w * TPOS

    tables = (emb0, emb1, emb2, emb3)
    r0 = jnp.minimum(sid * ROWS_PER_SUB, VOCAB - ROWS_PER_SUB)
    for c in range(N_FREQ):
        pltpu.sync_copy(
            tables[c].at[pl.ds(r0 * CHAR_DIM, ROWS_PER_SUB * CHAR_DIM)],
            tbl_v[c])

    def stage_body(r, carry):
        for c in range(N_FREQ):
            for k in range(CHAR_DIM // LANES):
                v = tbl_v[c][pl.ds(r * CHAR_DIM + k * LANES, LANES)]
                cat_v[r, pl.ds(c * CHAR_DIM + k * LANES, LANES)] = v
        return carry

    lax.fori_loop(0, ROWS_PER_SUB, stage_body, 0)
    pltpu.sync_copy(cat_v, cat_sh.at[pl.ds(r0, ROWS_PER_SUB)])
    plsc.subcore_barrier()

    def tok0_of(u):
        tc, b = divmod(u, B)
        return b * T + t_base + tc * CHUNK

    def start_idx(u):
        return pltpu.async_copy(
            idx_hbm.at[pl.ds(tok0_of(u), CHUNK)], idx_v[u], si[u])

    def start_pos(tc):
        return pltpu.async_copy(
            pos_hbm.at[pl.ds(t_base + tc * CHUNK, CHUNK)], pos_v[tc], sp[tc])

    def start_gather(u):
        p = u % 2
        return pltpu.async_copy(cat_sh.at[idx_v[u]], rows_v[p], sg[p])

    def start_out(u):
        p = u % 2
        return pltpu.async_copy(
            rows_v[p], out_hbm.at[pl.ds(tok0_of(u), CHUNK)], so[p])

    di = []
    for u in range(NU):
        di.append(start_idx(u))
        if u == 0:
            dpos = [start_pos(0), start_pos(1)]
    di[0].wait()
    dg = [start_gather(0), None]
    dout = [None, None]
    dpos[0].wait()
    dpos[1].wait()

    for u in range(NU):
        p = u % 2
        q = 1 - p
        if u + 1 < NU:
            if dout[q] is not None:
                dout[q].wait()
                dout[q] = None
            di[u + 1].wait()
            dg[q] = start_gather(u + 1)
        dg[p].wait()

        rows = rows_v[p]
        pos = pos_v[u // B]

        def tok_body(i, c2, rows=rows, pos=pos):
            for k in range(OUT_DIM // LANES):
                v = pos[i, pl.ds(k * LANES, LANES)]
                plsc.addupdate(rows.at[i, pl.ds(k * LANES, LANES)], v)
            return c2

        lax.fori_loop(0, CHUNK, tok_body, 0)
        dout[p] = start_out(u)

    dout[0].wait()
    dout[1].wait()


def kernel(raw_char_indices, emb0, emb1, emb2, emb3, pos_table):
    idx = raw_char_indices.reshape(NTOK)
    out = _encode(idx, emb0.reshape(-1), emb1.reshape(-1), emb2.reshape(-1),
                  emb3.reshape(-1), pos_table)
    return out.reshape(B, T, OUT_DIM)


# speedup vs baseline: 1.0375x; 1.0375x over previous
import functools

import jax
import jax.numpy as jnp
from jax import lax
from jax.experimental import pallas as pl
from jax.experimental.pallas import tpu as pltpu
from jax.experimental.pallas import tpu_sc as plsc

B, T = 4, 8192
VOCAB, CHAR_DIM, N_FREQ = 1000, 32, 4
OUT_DIM = CHAR_DIM * N_FREQ
NTOK = B * T
NUM_CORES, NUM_SUBCORES, LANES = 2, 16, 16
NW = NUM_CORES * NUM_SUBCORES
TPW = NTOK // NW
CHUNK = 128
TPOS = T // NW
NTC = TPOS // CHUNK
NU = NTC * B
ROWS_PER_SUB = 64

_mesh = plsc.VectorSubcoreMesh(core_axis_name="c", subcore_axis_name="s")


@functools.partial(
    pl.kernel,
    out_type=jax.ShapeDtypeStruct((NTOK, OUT_DIM), jnp.float32),
    mesh=_mesh,
    scratch_types=[
        [pltpu.VMEM((CHUNK,), jnp.int32) for _ in range(NU)],
        [pltpu.VMEM((CHUNK, OUT_DIM), jnp.float32) for _ in range(2)],
        [pltpu.VMEM((CHUNK, OUT_DIM), jnp.float32) for _ in range(2)],
        [pltpu.VMEM((ROWS_PER_SUB, CHAR_DIM), jnp.float32)
         for _ in range(N_FREQ)],
        pltpu.VMEM((ROWS_PER_SUB, OUT_DIM), jnp.float32),
        pltpu.VMEM_SHARED((VOCAB, OUT_DIM), jnp.float32),
        [pltpu.SemaphoreType.DMA for _ in range(NU)],
        [pltpu.SemaphoreType.DMA for _ in range(2)],
        [pltpu.SemaphoreType.DMA for _ in range(2)],
        [pltpu.SemaphoreType.DMA for _ in range(2)],
    ],
)
def _encode(idx_hbm, emb0, emb1, emb2, emb3, pos_hbm, out_hbm,
            idx_v, pos_v, rows_v, tbl_v, cat_v, cat_sh, si, sp, sg, so):
    sid = lax.axis_index("s")
    w = sid * NUM_CORES + lax.axis_index("c")
    t_base = w * TPOS

    tables = (emb0, emb1, emb2, emb3)
    r0 = jnp.minimum(sid * ROWS_PER_SUB, VOCAB - ROWS_PER_SUB)
    for c in range(N_FREQ):
        pltpu.sync_copy(tables[c].at[pl.ds(r0, ROWS_PER_SUB)], tbl_v[c])

    def stage_body(r, carry):
        for c in range(N_FREQ):
            for k in range(CHAR_DIM // LANES):
                v = tbl_v[c][r, pl.ds(k * LANES, LANES)]
                cat_v[r, pl.ds(c * CHAR_DIM + k * LANES, LANES)] = v
        return carry

    lax.fori_loop(0, ROWS_PER_SUB, stage_body, 0)
    pltpu.sync_copy(cat_v, cat_sh.at[pl.ds(r0, ROWS_PER_SUB)])
    plsc.subcore_barrier()

    def bt0_of(u):
        tc, b = divmod(u, B)
        return b, t_base + tc * CHUNK

    def tok0_of(u):
        b, t0 = bt0_of(u)
        return b * T + t0

    def start_idx(u):
        b, t0 = bt0_of(u)
        return pltpu.async_copy(
            idx_hbm.at[b, pl.ds(t0, CHUNK)], idx_v[u], si[u])

    def start_pos(tc):
        return pltpu.async_copy(
            pos_hbm.at[pl.ds(t_base + tc * CHUNK, CHUNK)], pos_v[tc], sp[tc])

    def start_gather(u):
        p = u % 2
        return pltpu.async_copy(cat_sh.at[idx_v[u]], rows_v[p], sg[p])

    def start_out(u):
        p = u % 2
        return pltpu.async_copy(
            rows_v[p], out_hbm.at[pl.ds(tok0_of(u), CHUNK)], so[p])

    di = []
    for u in range(NU):
        di.append(start_idx(u))
        if u == 0:
            dpos = [start_pos(0), start_pos(1)]
    di[0].wait()
    dg = [start_gather(0), None]
    dout = [None, None]
    dpos[0].wait()
    dpos[1].wait()

    for u in range(NU):
        p = u % 2
        q = 1 - p
        if u + 1 < NU:
            if dout[q] is not None:
                dout[q].wait()
                dout[q] = None
            di[u + 1].wait()
            dg[q] = start_gather(u + 1)
        dg[p].wait()

        rows = rows_v[p]
        pos = pos_v[u // B]

        def tok_body(i, c2, rows=rows, pos=pos):
            for k in range(OUT_DIM // LANES):
                v = pos[i, pl.ds(k * LANES, LANES)]
                plsc.addupdate(rows.at[i, pl.ds(k * LANES, LANES)], v)
            return c2

        lax.fori_loop(0, CHUNK, tok_body, 0)
        dout[p] = start_out(u)

    dout[0].wait()
    dout[1].wait()


def kernel(raw_char_indices, emb0, emb1, emb2, emb3, pos_table):
    out = _encode(raw_char_indices, emb0, emb1, emb2, emb3, pos_table)
    return out.reshape(B, T, OUT_DIM)
